# baseline (speedup 1.0000x reference)
import jax
import jax.numpy as jnp
import numpy as np
from jax.experimental import pallas as pl
from jax.experimental.pallas import tpu as pltpu

B, S, D = 1, 2048, 768
E = 10
TOP_K = 2
CITY_LEN = 10
CITY_EMB = 32
H = 768

_NEG = -1e30

_CDIMS = (((1,), (1,)), ((), ()))


def _threefry_bits():
    n = S * E
    rots = [[13, 15, 26, 6], [17, 29, 16, 24]]
    k0, k1 = np.uint32(0), np.uint32(42)
    ks = [k0, k1, k0 ^ k1 ^ np.uint32(0x1BD11BDA)]
    x0 = np.full(n, ks[0], np.uint32)
    x1 = (np.arange(n, dtype=np.uint32) + ks[1]).astype(np.uint32)
    for i in range(5):
        for r in rots[i % 2]:
            x0 = (x0 + x1).astype(np.uint32)
            x1 = ((x1 << np.uint32(r)) | (x1 >> np.uint32(32 - r)))
            x1 = (x1 ^ x0).astype(np.uint32)
        x0 = (x0 + ks[(i + 1) % 3]).astype(np.uint32)
        x1 = (x1 + ks[(i + 2) % 3] + np.uint32(i + 1)).astype(np.uint32)
    return (x0 ^ x1).reshape(S, E)


_NOISE_BITS = _threefry_bits()


def _noise():
    lo = np.nextafter(np.float32(-1), np.float32(0)).astype(np.float32)
    b = jnp.asarray(np.ascontiguousarray(_NOISE_BITS.T))
    u = jax.lax.bitcast_convert_type(
        (b >> np.uint32(9)) | np.uint32(0x3F800000), jnp.float32)
    u = u - np.float32(1)
    u = jnp.maximum(lo, u * (np.float32(1) - lo) + lo)
    return np.float32(np.sqrt(2)) * jax.lax.erf_inv(u)


def _fused_kernel(x_ref, dt_ref, dis_ref, rg_ref, ent_ref, city_ref,
                  cemb_ref, rw_ref, rb_ref, nw_ref, nb_ref, noise_ref,
                  w1_ref, b1_ref, w2_ref, b2_ref,
                  gate1_ref, out_ref, gating_ref, xb_ref):
    s = pl.program_id(0)

    @pl.when(s == 0)
    def _router():
        xb_ref[...] = x_ref[...].astype(jnp.bfloat16)
        eiota = jax.lax.broadcasted_iota(jnp.int32, (E, S), 0)

        coh = (jax.lax.broadcasted_iota(jnp.int32, (1, CITY_LEN), 1)
               == city_ref[0, 0]).astype(jnp.float32)
        ce = jnp.dot(coh, cemb_ref[...], preferred_element_type=jnp.float32)

        w = jnp.concatenate([rw_ref[...], nw_ref[...]], axis=0)

        def piece(f, c0, width):
            return jax.lax.dot_general(f, w[:, c0:c0 + width], _CDIMS,
                                       preferred_element_type=jnp.float32)

        o = piece(x_ref[...], 0, D)
        o += piece(ce, D, CITY_EMB)
        o += piece(dt_ref[...], D + CITY_EMB, D // 4)
        o += piece(dis_ref[...], D + CITY_EMB + D // 4, D // 4)
        o += piece(rg_ref[...], D + CITY_EMB + D // 2, D // 8)
        o += piece(ent_ref[...], D + CITY_EMB + D // 2 + D // 8, D // 8)
        ot = o.T

        logits = ot[:E] + rb_ref[...]
        nse = ot[E:] + nb_ref[...]
        std = jnp.maximum(nse, 0.0) + jnp.log1p(jnp.exp(-jnp.abs(nse)))
        noisy = logits + noise_ref[...] * std

        m1 = jnp.max(noisy, axis=0, keepdims=True)
        i1 = jnp.min(jnp.where(noisy == m1, eiota, 999), axis=0, keepdims=True)
        noisy2 = jnp.where(eiota == i1, _NEG, noisy)
        m2 = jnp.max(noisy2, axis=0, keepdims=True)
        i2 = jnp.min(jnp.where(noisy2 == m2, eiota, 999), axis=0, keepdims=True)
        eb = jnp.exp(m2 - m1)
        g1 = 1.0 / (1.0 + eb)
        g2 = eb * g1
        gating_ref[...] = (jnp.where(eiota == i1, g1, 0.0)
                           + jnp.where(eiota == i2, g2, 0.0)).T

        lm = jnp.max(logits, axis=0, keepdims=True)
        ex = jnp.exp(logits - lm)
        gate1_ref[...] = (ex / jnp.sum(ex, axis=0, keepdims=True)).T

    @pl.when(s > 0)
    def _expert():
        contrib = None
        for sub in range(2):
            e = (s - 1) * 2 + sub
            h = jnp.maximum(
                jnp.dot(xb_ref[...], w1_ref[sub].astype(jnp.bfloat16),
                        preferred_element_type=jnp.float32)
                + b1_ref[sub], 0.0)
            y = (jnp.dot(h.astype(jnp.bfloat16),
                         w2_ref[sub].astype(jnp.bfloat16),
                         preferred_element_type=jnp.float32)
                 + b2_ref[sub])
            eoh = (jax.lax.broadcasted_iota(jnp.int32, (E, 1), 0) == e
                   ).astype(jnp.float32)
            g = jnp.dot(gating_ref[...], eoh,
                        preferred_element_type=jnp.float32)
            contrib = y * g if contrib is None else contrib + y * g

        @pl.when(s == 1)
        def _():
            out_ref[...] = contrib

        @pl.when(s > 1)
        def _():
            out_ref[...] += contrib


def kernel(x, city, delta_t_info, delta_dis_info, delta_rg_info,
           delta_entropy_info, city_embeddings, route_W, route_b,
           noise_W, noise_b, W1, b1, W2, b2):
    x2d = x[0]
    noise = _noise()

    def wmap(s):
        e = jnp.maximum(s - 1, 0)
        return (e, 0, 0)

    gate1, out = pl.pallas_call(
        _fused_kernel,
        grid=(E // 2 + 1,),
        in_specs=[
            pl.BlockSpec((S, D), lambda s: (0, 0)),
            pl.BlockSpec((S, D // 4), lambda s: (0, 0)),
            pl.BlockSpec((S, D // 4), lambda s: (0, 0)),
            pl.BlockSpec((S, D // 8), lambda s: (0, 0)),
            pl.BlockSpec((S, D // 8), lambda s: (0, 0)),
            pl.BlockSpec((1, 1), lambda s: (0, 0)),
            pl.BlockSpec((CITY_LEN, CITY_EMB), lambda s: (0, 0)),
            pl.BlockSpec((E, D + CITY_EMB + 3 * D // 4),
                         lambda s: (0, 0)),
            pl.BlockSpec((E, 1), lambda s: (0, 0)),
            pl.BlockSpec((E, D + CITY_EMB + 3 * D // 4),
                         lambda s: (0, 0)),
            pl.BlockSpec((E, 1), lambda s: (0, 0)),
            pl.BlockSpec((E, S), lambda s: (0, 0)),
            pl.BlockSpec((2, D, H), wmap),
            pl.BlockSpec((2, 1, H), wmap),
            pl.BlockSpec((2, H, D), wmap),
            pl.BlockSpec((2, 1, D), wmap),
        ],
        out_specs=(pl.BlockSpec((S, E), lambda s: (0, 0)),
                   pl.BlockSpec((S, D), lambda s: (0, 0))),
        out_shape=(jax.ShapeDtypeStruct((S, E), jnp.float32),
                   jax.ShapeDtypeStruct((S, D), jnp.float32)),
        scratch_shapes=[pltpu.VMEM((S, E), jnp.float32),
                        pltpu.VMEM((S, D), jnp.bfloat16)],
    )(x2d, delta_t_info[0], delta_dis_info[0], delta_rg_info[0],
      delta_entropy_info[0], city.reshape(1, 1).astype(jnp.int32),
      city_embeddings, route_W, route_b.reshape(E, 1), noise_W,
      noise_b.reshape(E, 1), noise, W1, b1[:, None, :], W2, b2[:, None, :])

    return (out[None], gate1[None])


# speedup vs baseline: 1.0019x; 1.0019x over previous
import jax
import jax.numpy as jnp
import numpy as np
from jax.experimental import pallas as pl
from jax.experimental.pallas import tpu as pltpu

B, S, D = 1, 2048, 768
E = 10
TOP_K = 2
CITY_LEN = 10
CITY_EMB = 32
H = 768

_NEG = -1e30

_CDIMS = (((1,), (1,)), ((), ()))


def _threefry_bits():
    n = S * E
    rots = [[13, 15, 26, 6], [17, 29, 16, 24]]
    k0, k1 = np.uint32(0), np.uint32(42)
    ks = [k0, k1, k0 ^ k1 ^ np.uint32(0x1BD11BDA)]
    x0 = np.full(n, ks[0], np.uint32)
    x1 = (np.arange(n, dtype=np.uint32) + ks[1]).astype(np.uint32)
    for i in range(5):
        for r in rots[i % 2]:
            x0 = (x0 + x1).astype(np.uint32)
            x1 = ((x1 << np.uint32(r)) | (x1 >> np.uint32(32 - r)))
            x1 = (x1 ^ x0).astype(np.uint32)
        x0 = (x0 + ks[(i + 1) % 3]).astype(np.uint32)
        x1 = (x1 + ks[(i + 2) % 3] + np.uint32(i + 1)).astype(np.uint32)
    return (x0 ^ x1).reshape(S, E)


_NOISE_BITS = _threefry_bits()


def _noise():
    lo = np.nextafter(np.float32(-1), np.float32(0)).astype(np.float32)
    b = jnp.asarray(np.ascontiguousarray(_NOISE_BITS.T))
    u = jax.lax.bitcast_convert_type(
        (b >> np.uint32(9)) | np.uint32(0x3F800000), jnp.float32)
    u = u - np.float32(1)
    u = jnp.maximum(lo, u * (np.float32(1) - lo) + lo)
    return np.float32(np.sqrt(2)) * jax.lax.erf_inv(u)


def _fused_kernel(x_ref, dt_ref, dis_ref, rg_ref, ent_ref, city_ref,
                  cemb_ref, rw_ref, rb_ref, nw_ref, nb_ref, noise_ref,
                  w1_ref, b1_ref, w2_ref, b2_ref,
                  gate1_ref, out_ref, gating_ref, xb_ref):
    s = pl.program_id(0)

    @pl.when(s == 0)
    def _router():
        xb_ref[...] = x_ref[...].astype(jnp.bfloat16)
        eiota = jax.lax.broadcasted_iota(jnp.int32, (E, S), 0)

        coh = (jax.lax.broadcasted_iota(jnp.int32, (1, CITY_LEN), 1)
               == city_ref[0, 0]).astype(jnp.float32)
        ce = jnp.dot(coh, cemb_ref[...], preferred_element_type=jnp.float32)

        w = jnp.concatenate([rw_ref[...], nw_ref[...]], axis=0)

        def piece(f, c0, width):
            return jax.lax.dot_general(f, w[:, c0:c0 + width], _CDIMS,
                                       preferred_element_type=jnp.float32)

        o = piece(x_ref[...], 0, D)
        o += piece(ce, D, CITY_EMB)
        o += piece(dt_ref[...], D + CITY_EMB, D // 4)
        o += piece(dis_ref[...], D + CITY_EMB + D // 4, D // 4)
        o += piece(rg_ref[...], D + CITY_EMB + D // 2, D // 8)
        o += piece(ent_ref[...], D + CITY_EMB + D // 2 + D // 8, D // 8)
        ot = o.T

        logits = ot[:E] + rb_ref[...]
        nse = ot[E:] + nb_ref[...]
        std = jnp.maximum(nse, 0.0) + jnp.log1p(jnp.exp(-jnp.abs(nse)))
        noisy = logits + noise_ref[...] * std

        m1 = jnp.max(noisy, axis=0, keepdims=True)
        i1 = jnp.min(jnp.where(noisy == m1, eiota, 999), axis=0, keepdims=True)
        noisy2 = jnp.where(eiota == i1, _NEG, noisy)
        m2 = jnp.max(noisy2, axis=0, keepdims=True)
        i2 = jnp.min(jnp.where(noisy2 == m2, eiota, 999), axis=0, keepdims=True)
        eb = jnp.exp(m2 - m1)
        g1 = 1.0 / (1.0 + eb)
        g2 = eb * g1
        gating_ref[...] = (jnp.where(eiota == i1, g1, 0.0)
                           + jnp.where(eiota == i2, g2, 0.0)).T

        lm = jnp.max(logits, axis=0, keepdims=True)
        ex = jnp.exp(logits - lm)
        gate1_ref[...] = (ex / jnp.sum(ex, axis=0, keepdims=True)).T

    @pl.when(s > 0)
    def _expert():
        contrib = None
        for sub in range(2):
            e = (s - 1) * 2 + sub
            h = jnp.maximum(
                jnp.dot(xb_ref[...], w1_ref[sub].astype(jnp.bfloat16),
                        preferred_element_type=jnp.float32)
                + b1_ref[sub], 0.0)
            y = (jnp.dot(h.astype(jnp.bfloat16),
                         w2_ref[sub].astype(jnp.bfloat16),
                         preferred_element_type=jnp.float32)
                 + b2_ref[sub])
            eoh = (jax.lax.broadcasted_iota(jnp.int32, (E, 1), 0) == e
                   ).astype(jnp.float32)
            g = jnp.dot(gating_ref[...], eoh,
                        preferred_element_type=jnp.float32)
            contrib = y * g if contrib is None else contrib + y * g

        @pl.when(s == 1)
        def _():
            out_ref[...] = contrib

        @pl.when(s > 1)
        def _():
            out_ref[...] += contrib


def kernel(x, city, delta_t_info, delta_dis_info, delta_rg_info,
           delta_entropy_info, city_embeddings, route_W, route_b,
           noise_W, noise_b, W1, b1, W2, b2):
    x2d = x[0]
    with jax.ensure_compile_time_eval():
        noise = _noise()

    def wmap(s):
        e = jnp.maximum(s - 1, 0)
        return (e, 0, 0)

    gate1, out = pl.pallas_call(
        _fused_kernel,
        grid=(E // 2 + 1,),
        in_specs=[
            pl.BlockSpec((S, D), lambda s: (0, 0)),
            pl.BlockSpec((S, D // 4), lambda s: (0, 0)),
            pl.BlockSpec((S, D // 4), lambda s: (0, 0)),
            pl.BlockSpec((S, D // 8), lambda s: (0, 0)),
            pl.BlockSpec((S, D // 8), lambda s: (0, 0)),
            pl.BlockSpec((1, 1), lambda s: (0, 0)),
            pl.BlockSpec((CITY_LEN, CITY_EMB), lambda s: (0, 0)),
            pl.BlockSpec((E, D + CITY_EMB + 3 * D // 4),
                         lambda s: (0, 0)),
            pl.BlockSpec((E, 1), lambda s: (0, 0)),
            pl.BlockSpec((E, D + CITY_EMB + 3 * D // 4),
                         lambda s: (0, 0)),
            pl.BlockSpec((E, 1), lambda s: (0, 0)),
            pl.BlockSpec((E, S), lambda s: (0, 0)),
            pl.BlockSpec((2, D, H), wmap),
            pl.BlockSpec((2, 1, H), wmap),
            pl.BlockSpec((2, H, D), wmap),
            pl.BlockSpec((2, 1, D), wmap),
        ],
        out_specs=(pl.BlockSpec((S, E), lambda s: (0, 0)),
                   pl.BlockSpec((S, D), lambda s: (0, 0))),
        out_shape=(jax.ShapeDtypeStruct((S, E), jnp.float32),
                   jax.ShapeDtypeStruct((S, D), jnp.float32)),
        scratch_shapes=[pltpu.VMEM((S, E), jnp.float32),
                        pltpu.VMEM((S, D), jnp.bfloat16)],
    )(x2d, delta_t_info[0], delta_dis_info[0], delta_rg_info[0],
      delta_entropy_info[0], city.reshape(1, 1).astype(jnp.int32),
      city_embeddings, route_W, route_b.reshape(E, 1), noise_W,
      noise_b.reshape(E, 1), noise, W1, b1[:, None, :], W2, b2[:, None, :])

    return (out[None], gate1[None])
